# baseline (speedup 1.0000x reference)
import functools

import jax
import jax.numpy as jnp
from jax import lax
from jax.experimental import pallas as pl
from jax.experimental.pallas import tpu as pltpu
from jax.experimental.pallas import tpu_sc as plsc

N = 10000
D = 128
E = 320000
EPS = 1e-5

NC, NS = 2, 16
NW = NC * NS
K = 80
CHUNKS = E // K
CPW = CHUNKS // NW
NBUF = 5
MLAG = 2
RPS = N // NS
NPAD1 = 10240
SEG = NPAD1 // NS

BN = 1000
GRID = N // BN


def _mesh():
    return plsc.VectorSubcoreMesh(
        core_axis_name="c", subcore_axis_name="s", num_cores=NC, num_subcores=NS
    )


_sc_params = pltpu.CompilerParams(use_tc_tiling_on_sc=False)


@functools.partial(
    pl.kernel,
    out_type=jax.ShapeDtypeStruct((NC, NPAD1), jnp.float32),
    mesh=_mesh(),
    compiler_params=_sc_params,
    scratch_types=[
        pltpu.VMEM((CPW, K), jnp.int32),
        pltpu.VMEM((K,), jnp.float32),
        pltpu.VMEM((SEG,), jnp.float32),
        pltpu.VMEM_SHARED((NPAD1,), jnp.float32),
    ],
)
def _deg_kernel(col_hbm, out_hbm, idx_v, ones_v, zb_v, deg_sh):
    c = lax.axis_index("c")
    s = lax.axis_index("s")
    wid = s * NC + c

    def fill_z(i, carry):
        zb_v[pl.ds(i * 16, 16)] = jnp.zeros((16,), jnp.float32)
        return carry

    lax.fori_loop(0, SEG // 16, fill_z, 0)

    def fill_o(i, carry):
        ones_v[pl.ds(i * 16, 16)] = jnp.full((16,), 1.0, jnp.float32)
        return carry

    lax.fori_loop(0, K // 16, fill_o, 0)

    pltpu.sync_copy(zb_v, deg_sh.at[pl.ds(s * SEG, SEG)])
    pltpu.sync_copy(col_hbm.at[pl.ds(wid * CPW, CPW)], idx_v)
    plsc.subcore_barrier()

    def body(j, carry):
        pltpu.sync_copy(ones_v, deg_sh.at[idx_v.at[j]], add=True)
        return carry

    lax.fori_loop(0, CPW, body, 0)
    plsc.subcore_barrier()
    pltpu.sync_copy(deg_sh.at[pl.ds(s * SEG, SEG)], out_hbm.at[c, pl.ds(s * SEG, SEG)])


DH = D // 2
CPC = CHUNKS // NS


@functools.partial(
    pl.kernel,
    out_type=jax.ShapeDtypeStruct((NC, N, DH), jnp.float32),
    mesh=_mesh(),
    compiler_params=_sc_params,
    scratch_types=[
        pltpu.VMEM((CPC, K), jnp.int32),
        pltpu.VMEM((CPC, K), jnp.int32),
        pltpu.VMEM((125, DH), jnp.float32),
        [pltpu.VMEM((K, DH), jnp.float32) for _ in range(NBUF)],
        [pltpu.SemaphoreType.DMA for _ in range(NBUF)],
        [pltpu.SemaphoreType.DMA for _ in range(NBUF)],
        pltpu.VMEM_SHARED((N, DH), jnp.float32),
    ],
)
def _conv_kernel(h2_hbm, row_hbm, col_hbm, out_hbm, idxr_v, idxc_v, zb_v, bufs,
                 sems, sems2, acc_sh):
    c = lax.axis_index("c")
    s = lax.axis_index("s")
    h2c = h2_hbm.at[c]

    def fill_z(i, carry):
        for g in range(DH // 16):
            zb_v[i, pl.ds(g * 16, 16)] = jnp.zeros((16,), jnp.float32)
        return carry

    lax.fori_loop(0, 125, fill_z, 0)
    for t in range(RPS // 125):
        pltpu.sync_copy(zb_v, acc_sh.at[pl.ds(s * RPS + t * 125, 125)])

    pltpu.sync_copy(row_hbm.at[pl.ds(s * CPC, CPC)], idxr_v)
    pltpu.sync_copy(col_hbm.at[pl.ds(s * CPC, CPC)], idxc_v)
    plsc.subcore_barrier()

    for b in range(NBUF - MLAG):
        pltpu.async_copy(h2c.at[idxr_v.at[b]], bufs[b], sems[b])

    def outer(i, carry):
        for b in range(NBUF):
            j = i * NBUF + b
            bw = (b + NBUF - MLAG) % NBUF

            @pl.when(j >= MLAG)
            def _drain_scatter():
                pltpu.make_async_copy(
                    bufs[bw], acc_sh.at[idxc_v.at[j - MLAG]], sems2[bw]).wait()

            @pl.when(j + NBUF - MLAG < CPC)
            def _start_gather():
                pltpu.async_copy(
                    h2c.at[idxr_v.at[j + NBUF - MLAG]], bufs[bw], sems[bw])

            pltpu.make_async_copy(h2c.at[idxr_v.at[j]], bufs[b], sems[b]).wait()
            pltpu.async_copy(bufs[b], acc_sh.at[idxc_v.at[j]], sems2[b],
                             add=True)

        return carry

    lax.fori_loop(0, CPC // NBUF, outer, 0)
    for t in range(MLAG):
        jj = CPC - MLAG + t
        bb = jj % NBUF
        pltpu.make_async_copy(bufs[bb], acc_sh.at[idxc_v.at[jj]],
                              sems2[bb]).wait()
    plsc.subcore_barrier()
    pltpu.sync_copy(acc_sh.at[pl.ds(s * RPS, RPS)], out_hbm.at[c, pl.ds(s * RPS, RPS)])


def _dis_from(degp_ref):
    deg = degp_ref[0, 0, :] + degp_ref[0, 1, :] + 2.0
    return lax.rsqrt(deg)


def _split_write(out_ref, h2):
    out_ref[0, :, :] = h2[:, :DH]
    out_ref[1, :, :] = h2[:, DH:]


def _mm_scale_body(degp_ref, x_ref, w_ref, h2_ref):
    dis = _dis_from(degp_ref)
    h = jnp.dot(x_ref[...], w_ref[...], preferred_element_type=jnp.float32)
    _split_write(h2_ref, h * dis[:, None])


def _combine_body(degp_ref, acc_ref, h2_ref, b_ref, pre_ref, st_ref):
    i = pl.program_id(0)
    dis = _dis_from(degp_ref)[:, None]
    tot = jnp.concatenate([acc_ref[0], acc_ref[1]], axis=1)
    h2 = jnp.concatenate([h2_ref[0], h2_ref[1]], axis=1)
    pre = dis * tot + 2.0 * dis * h2 + b_ref[...]
    pre_ref[...] = pre
    st = jnp.concatenate(
        [jnp.sum(pre, axis=0, keepdims=True),
         jnp.sum(pre * pre, axis=0, keepdims=True)], axis=0)

    @pl.when(i == 0)
    def _init():
        st_ref[...] = st

    @pl.when(i > 0)
    def _acc():
        st_ref[...] += st


def _bn_relu_res(pre_ref, st_ref, g_ref, be_ref, res_ref):
    m = st_ref[0:1, :] * (1.0 / N)
    v = st_ref[1:2, :] * (1.0 / N) - m * m
    inv = lax.rsqrt(v + EPS)
    return (
        jnp.maximum((pre_ref[...] - m) * inv * g_ref[...] + be_ref[...], 0.0)
        + res_ref[...]
    )


def _mid_body(degp_ref, pre_ref, st_ref, g_ref, be_ref, res_ref, w_ref,
              y_ref, h2_ref):
    y = _bn_relu_res(pre_ref, st_ref, g_ref, be_ref, res_ref)
    y_ref[...] = y
    dis = _dis_from(degp_ref)
    mm = jnp.dot(y, w_ref[...], preferred_element_type=jnp.float32)
    _split_write(h2_ref, mm * dis[:, None])


def _head_body(pre_ref, st_ref, g_ref, be_ref, res_ref, w_ref, wb_ref, out_ref):
    z = _bn_relu_res(pre_ref, st_ref, g_ref, be_ref, res_ref)
    out_ref[...] = (
        jnp.dot(z, w_ref[...], preferred_element_type=jnp.float32) + wb_ref[...]
    )


_seq = pltpu.CompilerParams(dimension_semantics=("arbitrary",))

_row_spec = pl.BlockSpec((BN, D), lambda i: (i, 0))
_degp_spec = pl.BlockSpec((1, 2, BN), lambda i: (i, 0, 0))
_full_spec = pl.BlockSpec((D, D), lambda i: (0, 0))
_vec_spec = pl.BlockSpec((1, D), lambda i: (0, 0))
_st_spec = pl.BlockSpec((2, D), lambda i: (0, 0))
_split_spec = pl.BlockSpec((2, BN, DH), lambda i: (0, i, 0))
_split_shape = jax.ShapeDtypeStruct((2, N, DH), jnp.float32)

_mm_scale = pl.pallas_call(
    _mm_scale_body,
    grid=(GRID,),
    in_specs=[_degp_spec, _row_spec, _full_spec],
    out_specs=_split_spec,
    out_shape=_split_shape,
    compiler_params=_seq,
)

_combine = pl.pallas_call(
    _combine_body,
    grid=(GRID,),
    in_specs=[_degp_spec, _split_spec, _split_spec, _vec_spec],
    out_specs=[_row_spec, _st_spec],
    out_shape=[jax.ShapeDtypeStruct((N, D), jnp.float32),
               jax.ShapeDtypeStruct((2, D), jnp.float32)],
    compiler_params=_seq,
)

_mid = pl.pallas_call(
    _mid_body,
    grid=(GRID,),
    in_specs=[_degp_spec, _row_spec, _st_spec, _vec_spec, _vec_spec, _row_spec,
              _full_spec],
    out_specs=[_row_spec, _split_spec],
    out_shape=[jax.ShapeDtypeStruct((N, D), jnp.float32), _split_shape],
    compiler_params=_seq,
)

_head = pl.pallas_call(
    _head_body,
    grid=(GRID,),
    in_specs=[_row_spec, _st_spec, _vec_spec, _vec_spec, _row_spec, _full_spec,
              _vec_spec],
    out_specs=_row_spec,
    out_shape=jax.ShapeDtypeStruct((N, D), jnp.float32),
    compiler_params=_seq,
)


def kernel(x, edge_index, W1, b1, g1, be1, W2, b2, g2, be2, Wh, bh):
    row2 = edge_index[0].reshape(CHUNKS, K)
    col2 = edge_index[1].reshape(CHUNKS, K)

    degp = _deg_kernel(col2)[:, :N]
    degp = degp.reshape(2, GRID, BN).transpose(1, 0, 2)
    h2 = _mm_scale(degp, x, W1)
    acc1 = _conv_kernel(h2, row2, col2)
    pre1, st1 = _combine(degp, acc1, h2, b1.reshape(1, D))
    y1, h2b = _mid(degp, pre1, st1, g1.reshape(1, D), be1.reshape(1, D), x, W2)
    acc2 = _conv_kernel(h2b, row2, col2)
    pre2, st2 = _combine(degp, acc2, h2b, b2.reshape(1, D))
    return _head(pre2, st2, g2.reshape(1, D), be2.reshape(1, D), y1, Wh,
                 bh.reshape(1, D))


# speedup vs baseline: 1.0181x; 1.0181x over previous
import functools

import jax
import jax.numpy as jnp
from jax import lax
from jax.experimental import pallas as pl
from jax.experimental.pallas import tpu as pltpu
from jax.experimental.pallas import tpu_sc as plsc

N = 10000
D = 128
E = 320000
EPS = 1e-5

NC, NS = 2, 16
NW = NC * NS
K = 80
CHUNKS = E // K
CPW = CHUNKS // NW
NBUF = 5
MLAG = 2
RPS = N // NS
NPAD1 = 10240
SEG = NPAD1 // NS

BN = 1000
GRID = N // BN


def _mesh():
    return plsc.VectorSubcoreMesh(
        core_axis_name="c", subcore_axis_name="s", num_cores=NC, num_subcores=NS
    )


_sc_params = pltpu.CompilerParams(use_tc_tiling_on_sc=False)


@functools.partial(
    pl.kernel,
    out_type=jax.ShapeDtypeStruct((NC, NPAD1), jnp.float32),
    mesh=_mesh(),
    compiler_params=_sc_params,
    scratch_types=[
        pltpu.VMEM((CPW, K), jnp.int32),
        pltpu.VMEM((K,), jnp.float32),
        pltpu.VMEM((SEG,), jnp.float32),
        pltpu.VMEM_SHARED((NPAD1,), jnp.float32),
    ],
)
def _deg_kernel(col_hbm, out_hbm, idx_v, ones_v, zb_v, deg_sh):
    c = lax.axis_index("c")
    s = lax.axis_index("s")
    wid = s * NC + c

    def fill_z(i, carry):
        zb_v[pl.ds(i * 16, 16)] = jnp.zeros((16,), jnp.float32)
        return carry

    lax.fori_loop(0, SEG // 16, fill_z, 0)

    def fill_o(i, carry):
        ones_v[pl.ds(i * 16, 16)] = jnp.full((16,), 1.0, jnp.float32)
        return carry

    lax.fori_loop(0, K // 16, fill_o, 0)

    pltpu.sync_copy(zb_v, deg_sh.at[pl.ds(s * SEG, SEG)])
    pltpu.sync_copy(col_hbm.at[pl.ds(wid * CPW, CPW)], idx_v)
    plsc.subcore_barrier()

    def body(j, carry):
        pltpu.sync_copy(ones_v, deg_sh.at[idx_v.at[j]], add=True)
        return carry

    lax.fori_loop(0, CPW, body, 0)
    plsc.subcore_barrier()
    pltpu.sync_copy(deg_sh.at[pl.ds(s * SEG, SEG)], out_hbm.at[c, pl.ds(s * SEG, SEG)])


DH = D // 2
CPC = CHUNKS // NS


@functools.partial(
    pl.kernel,
    out_type=jax.ShapeDtypeStruct((NC, N, DH), jnp.float32),
    mesh=_mesh(),
    compiler_params=_sc_params,
    scratch_types=[
        pltpu.VMEM((CPC, K), jnp.int32),
        pltpu.VMEM((CPC, K), jnp.int32),
        pltpu.VMEM((125, DH), jnp.float32),
        [pltpu.VMEM((K, DH), jnp.float32) for _ in range(NBUF)],
        [pltpu.SemaphoreType.DMA for _ in range(NBUF)],
        [pltpu.SemaphoreType.DMA for _ in range(NBUF)],
        pltpu.VMEM_SHARED((N, DH), jnp.float32),
    ],
)
def _conv_kernel(h2_hbm, row_hbm, col_hbm, out_hbm, idxr_v, idxc_v, zb_v, bufs,
                 sems, sems2, acc_sh):
    c = lax.axis_index("c")
    s = lax.axis_index("s")
    h2c = h2_hbm.at[c]

    def fill_z(i, carry):
        for g in range(DH // 16):
            zb_v[i, pl.ds(g * 16, 16)] = jnp.zeros((16,), jnp.float32)
        return carry

    lax.fori_loop(0, 125, fill_z, 0)
    for t in range(RPS // 125):
        pltpu.sync_copy(zb_v, acc_sh.at[pl.ds(s * RPS + t * 125, 125)])

    pltpu.sync_copy(row_hbm.at[pl.ds(s * CPC, CPC)], idxr_v)
    pltpu.sync_copy(col_hbm.at[pl.ds(s * CPC, CPC)], idxc_v)
    plsc.subcore_barrier()

    for b in range(NBUF - MLAG):
        pltpu.async_copy(h2c.at[idxr_v.at[b]], bufs[b], sems[b])

    def outer(i, carry):
        for b in range(NBUF):
            j = i * NBUF + b
            bw = (b + NBUF - MLAG) % NBUF

            @pl.when(j >= MLAG)
            def _drain_scatter():
                pltpu.make_async_copy(
                    bufs[bw], acc_sh.at[idxc_v.at[j - MLAG]], sems2[bw]).wait()

            @pl.when(j + NBUF - MLAG < CPC)
            def _start_gather():
                pltpu.async_copy(
                    h2c.at[idxr_v.at[j + NBUF - MLAG]], bufs[bw], sems[bw])

            pltpu.make_async_copy(h2c.at[idxr_v.at[j]], bufs[b], sems[b]).wait()
            pltpu.async_copy(bufs[b], acc_sh.at[idxc_v.at[j]], sems2[b],
                             add=True)

        return carry

    lax.fori_loop(0, CPC // NBUF, outer, 0)
    for t in range(MLAG):
        jj = CPC - MLAG + t
        bb = jj % NBUF
        pltpu.make_async_copy(bufs[bb], acc_sh.at[idxc_v.at[jj]],
                              sems2[bb]).wait()
    plsc.subcore_barrier()
    pltpu.sync_copy(acc_sh.at[pl.ds(s * RPS, RPS)], out_hbm.at[c, pl.ds(s * RPS, RPS)])


def _dis_from(degp_ref):
    deg = degp_ref[0, 0, :] + degp_ref[0, 1, :] + 2.0
    return lax.rsqrt(deg)


def _split_write(out_ref, h2):
    out_ref[0, :, :] = h2[:, :DH]
    out_ref[1, :, :] = h2[:, DH:]


def _mm_scale_body(degp_ref, x_ref, w_ref, h2_ref):
    dis = _dis_from(degp_ref)
    h = jnp.dot(x_ref[...], w_ref[...], preferred_element_type=jnp.float32)
    _split_write(h2_ref, h * dis[:, None])


def _block_phase0(degp_ref, acc_ref, h2_ref, b_ref, pre_s, st_s, j):
    dis = _dis_from(degp_ref)[:, None]
    tot = jnp.concatenate([acc_ref[0], acc_ref[1]], axis=1)
    h2 = jnp.concatenate([h2_ref[0], h2_ref[1]], axis=1)
    pre = dis * tot + 2.0 * dis * h2 + b_ref[...]
    pre_s[j] = pre
    st = jnp.concatenate(
        [jnp.sum(pre, axis=0, keepdims=True),
         jnp.sum(pre * pre, axis=0, keepdims=True)], axis=0)

    @pl.when(j == 0)
    def _init():
        st_s[...] = st

    @pl.when(j > 0)
    def _acc():
        st_s[...] += st


def _bn_relu_res(pre, st_s, g_ref, be_ref, res_ref):
    m = st_s[0:1, :] * (1.0 / N)
    v = st_s[1:2, :] * (1.0 / N) - m * m
    inv = lax.rsqrt(v + EPS)
    return (
        jnp.maximum((pre - m) * inv * g_ref[...] + be_ref[...], 0.0)
        + res_ref[...]
    )


def _mid_body(degp_ref, acc_ref, h2_ref, b_ref, g_ref, be_ref, res_ref, w_ref,
              y_ref, h2o_ref, pre_s, st_s):
    p = pl.program_id(0)
    j = pl.program_id(1)

    @pl.when(p == 0)
    def _phase0():
        _block_phase0(degp_ref, acc_ref, h2_ref, b_ref, pre_s, st_s, j)

    @pl.when(p == 1)
    def _phase1():
        y = _bn_relu_res(pre_s[j], st_s, g_ref, be_ref, res_ref)
        y_ref[...] = y
        dis = _dis_from(degp_ref)
        mm = jnp.dot(y, w_ref[...], preferred_element_type=jnp.float32)
        _split_write(h2o_ref, mm * dis[:, None])


def _head_body(degp_ref, acc_ref, h2_ref, b_ref, g_ref, be_ref, res_ref,
               w_ref, wb_ref, out_ref, pre_s, st_s):
    p = pl.program_id(0)
    j = pl.program_id(1)

    @pl.when(p == 0)
    def _phase0():
        _block_phase0(degp_ref, acc_ref, h2_ref, b_ref, pre_s, st_s, j)

    @pl.when(p == 1)
    def _phase1():
        z = _bn_relu_res(pre_s[j], st_s, g_ref, be_ref, res_ref)
        out_ref[...] = (
            jnp.dot(z, w_ref[...], preferred_element_type=jnp.float32)
            + wb_ref[...]
        )


_seq = pltpu.CompilerParams(dimension_semantics=("arbitrary",))

_row_spec = pl.BlockSpec((BN, D), lambda i: (i, 0))
_degp_spec = pl.BlockSpec((1, 2, BN), lambda i: (i, 0, 0))
_full_spec = pl.BlockSpec((D, D), lambda i: (0, 0))
_vec_spec = pl.BlockSpec((1, D), lambda i: (0, 0))
_st_spec = pl.BlockSpec((2, D), lambda i: (0, 0))
_split_spec = pl.BlockSpec((2, BN, DH), lambda i: (0, i, 0))
_split_shape = jax.ShapeDtypeStruct((2, N, DH), jnp.float32)

_mm_scale = pl.pallas_call(
    _mm_scale_body,
    grid=(GRID,),
    in_specs=[_degp_spec, _row_spec, _full_spec],
    out_specs=_split_spec,
    out_shape=_split_shape,
    compiler_params=_seq,
)

_seq2 = pltpu.CompilerParams(dimension_semantics=("arbitrary", "arbitrary"))

_degp_spec2 = pl.BlockSpec((1, 2, BN), lambda p, j: (j, 0, 0))
_split_p0 = pl.BlockSpec((2, BN, DH), lambda p, j: (0, jnp.where(p == 0, j, 0), 0))
_row_p1 = pl.BlockSpec((BN, D), lambda p, j: (jnp.where(p == 1, j, 0), 0))
_split_out_p1 = pl.BlockSpec((2, BN, DH),
                             lambda p, j: (0, jnp.where(p == 1, j, 0), 0))
_vec_spec2 = pl.BlockSpec((1, D), lambda p, j: (0, 0))
_full_spec2 = pl.BlockSpec((D, D), lambda p, j: (0, 0))

_m_scratch = [pltpu.VMEM((GRID, BN, D), jnp.float32),
              pltpu.VMEM((2, D), jnp.float32)]

_mid = pl.pallas_call(
    _mid_body,
    grid=(2, GRID),
    in_specs=[_degp_spec2, _split_p0, _split_p0, _vec_spec2, _vec_spec2,
              _vec_spec2, _row_p1, _full_spec2],
    out_specs=[_row_p1, _split_out_p1],
    out_shape=[jax.ShapeDtypeStruct((N, D), jnp.float32), _split_shape],
    scratch_shapes=_m_scratch,
    compiler_params=_seq2,
)

_head = pl.pallas_call(
    _head_body,
    grid=(2, GRID),
    in_specs=[_degp_spec2, _split_p0, _split_p0, _vec_spec2, _vec_spec2,
              _vec_spec2, _row_p1, _full_spec2, _vec_spec2],
    out_specs=_row_p1,
    out_shape=jax.ShapeDtypeStruct((N, D), jnp.float32),
    scratch_shapes=_m_scratch,
    compiler_params=_seq2,
)


def kernel(x, edge_index, W1, b1, g1, be1, W2, b2, g2, be2, Wh, bh):
    row2 = edge_index[0].reshape(CHUNKS, K)
    col2 = edge_index[1].reshape(CHUNKS, K)

    degp = _deg_kernel(col2)[:, :N]
    degp = degp.reshape(2, GRID, BN).transpose(1, 0, 2)
    h2 = _mm_scale(degp, x, W1)
    acc1 = _conv_kernel(h2, row2, col2)
    y1, h2b = _mid(degp, acc1, h2, b1.reshape(1, D), g1.reshape(1, D),
                   be1.reshape(1, D), x, W2)
    acc2 = _conv_kernel(h2b, row2, col2)
    return _head(degp, acc2, h2b, b2.reshape(1, D), g2.reshape(1, D),
                 be2.reshape(1, D), y1, Wh, bh.reshape(1, D))


# speedup vs baseline: 1.2280x; 1.2061x over previous
import functools

import jax
import jax.numpy as jnp
from jax import lax
from jax.experimental import pallas as pl
from jax.experimental.pallas import tpu as pltpu
from jax.experimental.pallas import tpu_sc as plsc

N = 10000
D = 128
E = 320000
EPS = 1e-5

NC, NS = 2, 16
NW = NC * NS
K = 80
CHUNKS = E // K
CPW = CHUNKS // NW
NBUF = 5
MLAG = 2
RPS = N // NS
NPAD1 = 10240
SEG = NPAD1 // NS

BN = 1000
GRID = N // BN


def _mesh():
    return plsc.VectorSubcoreMesh(
        core_axis_name="c", subcore_axis_name="s", num_cores=NC, num_subcores=NS
    )


_sc_params = pltpu.CompilerParams(use_tc_tiling_on_sc=False)


@functools.partial(
    pl.kernel,
    out_type=jax.ShapeDtypeStruct((NC, NPAD1), jnp.float32),
    mesh=_mesh(),
    compiler_params=_sc_params,
    scratch_types=[
        pltpu.VMEM((CPW, K), jnp.int32),
        pltpu.VMEM((K,), jnp.float32),
        pltpu.VMEM((SEG,), jnp.float32),
        pltpu.VMEM_SHARED((NPAD1,), jnp.float32),
    ],
)
def _deg_kernel(col_hbm, out_hbm, idx_v, ones_v, zb_v, deg_sh):
    c = lax.axis_index("c")
    s = lax.axis_index("s")
    wid = s * NC + c

    def fill_z(i, carry):
        zb_v[pl.ds(i * 16, 16)] = jnp.zeros((16,), jnp.float32)
        return carry

    lax.fori_loop(0, SEG // 16, fill_z, 0)

    def fill_o(i, carry):
        ones_v[pl.ds(i * 16, 16)] = jnp.full((16,), 1.0, jnp.float32)
        return carry

    lax.fori_loop(0, K // 16, fill_o, 0)

    pltpu.sync_copy(zb_v, deg_sh.at[pl.ds(s * SEG, SEG)])
    pltpu.sync_copy(col_hbm.at[pl.ds(wid * CPW, CPW)], idx_v)
    plsc.subcore_barrier()

    def body(j, carry):
        pltpu.sync_copy(ones_v, deg_sh.at[idx_v.at[j]], add=True)
        return carry

    lax.fori_loop(0, CPW, body, 0)
    plsc.subcore_barrier()
    pltpu.sync_copy(deg_sh.at[pl.ds(s * SEG, SEG)], out_hbm.at[c, pl.ds(s * SEG, SEG)])


DH = D // 2
CPC = CHUNKS // NS


@functools.partial(
    pl.kernel,
    out_type=jax.ShapeDtypeStruct((NC, N, DH), jnp.bfloat16),
    mesh=_mesh(),
    compiler_params=_sc_params,
    scratch_types=[
        pltpu.VMEM((CPC, K), jnp.int32),
        pltpu.VMEM((CPC, K), jnp.int32),
        pltpu.VMEM((125, DH), jnp.bfloat16),
        [pltpu.VMEM((K, DH), jnp.bfloat16) for _ in range(NBUF)],
        [pltpu.SemaphoreType.DMA for _ in range(NBUF)],
        [pltpu.SemaphoreType.DMA for _ in range(NBUF)],
        pltpu.VMEM_SHARED((N, DH), jnp.bfloat16),
    ],
)
def _conv_kernel(h2_hbm, row_hbm, col_hbm, out_hbm, idxr_v, idxc_v, zb_v, bufs,
                 sems, sems2, acc_sh):
    c = lax.axis_index("c")
    s = lax.axis_index("s")
    h2c = h2_hbm.at[c]

    def fill_z(i, carry):
        for g in range(DH // 32):
            zb_v[i, pl.ds(g * 32, 32)] = jnp.zeros((32,), jnp.bfloat16)
        return carry

    lax.fori_loop(0, 125, fill_z, 0)
    for t in range(RPS // 125):
        pltpu.sync_copy(zb_v, acc_sh.at[pl.ds(s * RPS + t * 125, 125)])

    pltpu.sync_copy(row_hbm.at[pl.ds(s * CPC, CPC)], idxr_v)
    pltpu.sync_copy(col_hbm.at[pl.ds(s * CPC, CPC)], idxc_v)
    plsc.subcore_barrier()

    for b in range(NBUF - MLAG):
        pltpu.async_copy(h2c.at[idxr_v.at[b]], bufs[b], sems[b])

    def outer(i, carry):
        for b in range(NBUF):
            j = i * NBUF + b
            bw = (b + NBUF - MLAG) % NBUF

            @pl.when(j >= MLAG)
            def _drain_scatter():
                pltpu.make_async_copy(
                    bufs[bw], acc_sh.at[idxc_v.at[j - MLAG]], sems2[bw]).wait()

            @pl.when(j + NBUF - MLAG < CPC)
            def _start_gather():
                pltpu.async_copy(
                    h2c.at[idxr_v.at[j + NBUF - MLAG]], bufs[bw], sems[bw])

            pltpu.make_async_copy(h2c.at[idxr_v.at[j]], bufs[b], sems[b]).wait()
            pltpu.async_copy(bufs[b], acc_sh.at[idxc_v.at[j]], sems2[b],
                             add=True)

        return carry

    lax.fori_loop(0, CPC // NBUF, outer, 0)
    for t in range(MLAG):
        jj = CPC - MLAG + t
        bb = jj % NBUF
        pltpu.make_async_copy(bufs[bb], acc_sh.at[idxc_v.at[jj]],
                              sems2[bb]).wait()
    plsc.subcore_barrier()
    pltpu.sync_copy(acc_sh.at[pl.ds(s * RPS, RPS)], out_hbm.at[c, pl.ds(s * RPS, RPS)])


def _dis_from(degp_ref):
    deg = degp_ref[0, 0, :] + degp_ref[0, 1, :] + 2.0
    return lax.rsqrt(deg)


def _split_write(out_ref, h2):
    h2 = h2.astype(jnp.bfloat16)
    out_ref[0, :, :] = h2[:, :DH]
    out_ref[1, :, :] = h2[:, DH:]


def _mm_scale_body(degp_ref, x_ref, w_ref, h2_ref):
    dis = _dis_from(degp_ref)
    h = jnp.dot(x_ref[...], w_ref[...], preferred_element_type=jnp.float32)
    _split_write(h2_ref, h * dis[:, None])


def _block_phase0(degp_ref, acc_ref, h2_ref, b_ref, pre_s, st_s, j):
    dis = _dis_from(degp_ref)[:, None]
    tot = jnp.concatenate(
        [acc_ref[0], acc_ref[1]], axis=1).astype(jnp.float32)
    h2 = jnp.concatenate(
        [h2_ref[0], h2_ref[1]], axis=1).astype(jnp.float32)
    pre = dis * tot + 2.0 * dis * h2 + b_ref[...]
    pre_s[j] = pre
    st = jnp.concatenate(
        [jnp.sum(pre, axis=0, keepdims=True),
         jnp.sum(pre * pre, axis=0, keepdims=True)], axis=0)

    @pl.when(j == 0)
    def _init():
        st_s[...] = st

    @pl.when(j > 0)
    def _acc():
        st_s[...] += st


def _bn_relu_res(pre, st_s, g_ref, be_ref, res_ref):
    m = st_s[0:1, :] * (1.0 / N)
    v = st_s[1:2, :] * (1.0 / N) - m * m
    inv = lax.rsqrt(v + EPS)
    return (
        jnp.maximum((pre - m) * inv * g_ref[...] + be_ref[...], 0.0)
        + res_ref[...]
    )


def _mid_body(degp_ref, acc_ref, h2_ref, b_ref, g_ref, be_ref, res_ref, w_ref,
              y_ref, h2o_ref, pre_s, st_s):
    p = pl.program_id(0)
    j = pl.program_id(1)

    @pl.when(p == 0)
    def _phase0():
        _block_phase0(degp_ref, acc_ref, h2_ref, b_ref, pre_s, st_s, j)

    @pl.when(p == 1)
    def _phase1():
        y = _bn_relu_res(pre_s[j], st_s, g_ref, be_ref, res_ref)
        y_ref[...] = y
        dis = _dis_from(degp_ref)
        mm = jnp.dot(y, w_ref[...], preferred_element_type=jnp.float32)
        _split_write(h2o_ref, mm * dis[:, None])


def _head_body(degp_ref, acc_ref, h2_ref, b_ref, g_ref, be_ref, res_ref,
               w_ref, wb_ref, out_ref, pre_s, st_s):
    p = pl.program_id(0)
    j = pl.program_id(1)

    @pl.when(p == 0)
    def _phase0():
        _block_phase0(degp_ref, acc_ref, h2_ref, b_ref, pre_s, st_s, j)

    @pl.when(p == 1)
    def _phase1():
        z = _bn_relu_res(pre_s[j], st_s, g_ref, be_ref, res_ref)
        out_ref[...] = (
            jnp.dot(z, w_ref[...], preferred_element_type=jnp.float32)
            + wb_ref[...]
        )


_seq = pltpu.CompilerParams(dimension_semantics=("arbitrary",))

_row_spec = pl.BlockSpec((BN, D), lambda i: (i, 0))
_degp_spec = pl.BlockSpec((1, 2, BN), lambda i: (i, 0, 0))
_full_spec = pl.BlockSpec((D, D), lambda i: (0, 0))
_vec_spec = pl.BlockSpec((1, D), lambda i: (0, 0))
_st_spec = pl.BlockSpec((2, D), lambda i: (0, 0))
_split_spec = pl.BlockSpec((2, BN, DH), lambda i: (0, i, 0))
_split_shape = jax.ShapeDtypeStruct((2, N, DH), jnp.bfloat16)

_mm_scale = pl.pallas_call(
    _mm_scale_body,
    grid=(GRID,),
    in_specs=[_degp_spec, _row_spec, _full_spec],
    out_specs=_split_spec,
    out_shape=_split_shape,
    compiler_params=_seq,
)

_seq2 = pltpu.CompilerParams(dimension_semantics=("arbitrary", "arbitrary"))

_degp_spec2 = pl.BlockSpec((1, 2, BN), lambda p, j: (j, 0, 0))
_split_p0 = pl.BlockSpec((2, BN, DH), lambda p, j: (0, jnp.where(p == 0, j, 0), 0))
_row_p1 = pl.BlockSpec((BN, D), lambda p, j: (jnp.where(p == 1, j, 0), 0))
_split_out_p1 = pl.BlockSpec((2, BN, DH),
                             lambda p, j: (0, jnp.where(p == 1, j, 0), 0))
_vec_spec2 = pl.BlockSpec((1, D), lambda p, j: (0, 0))
_full_spec2 = pl.BlockSpec((D, D), lambda p, j: (0, 0))

_m_scratch = [pltpu.VMEM((GRID, BN, D), jnp.float32),
              pltpu.VMEM((2, D), jnp.float32)]

_mid = pl.pallas_call(
    _mid_body,
    grid=(2, GRID),
    in_specs=[_degp_spec2, _split_p0, _split_p0, _vec_spec2, _vec_spec2,
              _vec_spec2, _row_p1, _full_spec2],
    out_specs=[_row_p1, _split_out_p1],
    out_shape=[jax.ShapeDtypeStruct((N, D), jnp.float32), _split_shape],
    scratch_shapes=_m_scratch,
    compiler_params=_seq2,
)

_head = pl.pallas_call(
    _head_body,
    grid=(2, GRID),
    in_specs=[_degp_spec2, _split_p0, _split_p0, _vec_spec2, _vec_spec2,
              _vec_spec2, _row_p1, _full_spec2, _vec_spec2],
    out_specs=_row_p1,
    out_shape=jax.ShapeDtypeStruct((N, D), jnp.float32),
    scratch_shapes=_m_scratch,
    compiler_params=_seq2,
)


def kernel(x, edge_index, W1, b1, g1, be1, W2, b2, g2, be2, Wh, bh):
    row2 = edge_index[0].reshape(CHUNKS, K)
    col2 = edge_index[1].reshape(CHUNKS, K)

    degp = _deg_kernel(col2)[:, :N]
    degp = degp.reshape(2, GRID, BN).transpose(1, 0, 2)
    h2bf = _mm_scale(degp, x, W1)
    acc1 = _conv_kernel(h2bf, row2, col2)
    y1, h2bbf = _mid(degp, acc1, h2bf, b1.reshape(1, D), g1.reshape(1, D),
                     be1.reshape(1, D), x, W2)
    acc2 = _conv_kernel(h2bbf, row2, col2)
    return _head(degp, acc2, h2bbf, b2.reshape(1, D), g2.reshape(1, D),
                 be2.reshape(1, D), y1, Wh, bh.reshape(1, D))
